# Initial kernel scaffold; baseline (speedup 1.0000x reference)
import jax
import jax.numpy as jnp
from jax.experimental import pallas as pl


def kernel(edge_index, edgeskip_index, nf, ef, W_s2r, b_s2r, aw_s2r, ab_s2r, W_o2r, b_o2r, aw_o2r, ab_o2r, W_r2s, b_r2s, aw_r2s, ab_r2s, W_r2o, b_r2o, aw_r2o, ab_r2o, W_skip, b_skip, aw_skip, ab_skip):
    raise NotImplementedError("write your pallas kernel here")


# speedup vs baseline: 1.0016x; 1.0016x over previous
import jax
import jax.numpy as jnp
from jax.experimental import pallas as pl
from jax.experimental.pallas import tpu as pltpu

N = 4096
E = 65536
D = 256
TOPK = 20


def _leaky(x):
    return jax.nn.leaky_relu(x, 0.2)


def _seg_softmax(e, idx, n):
    m = jax.ops.segment_max(e, idx, num_segments=n)
    out = jnp.exp(e - m[idx])
    s = jax.ops.segment_sum(out, idx, num_segments=n)
    return out / (s[idx] + 1e-16)


def _purify(e, ei, n, j, k):
    ev = jax.lax.stop_gradient(e)[:, 0]
    mask = jnp.full((n, n), -1e15, dtype=jnp.float32).at[ei[0], ei[1]].set(ev)
    if j == 1:
        thr = jax.lax.top_k(mask, k)[0].min(axis=1).reshape(-1, 1)
    else:
        thr = jax.lax.top_k(mask.T, k)[0].min(axis=1).reshape(1, -1)
    mask = jnp.where(mask <= thr, -1e15, mask)
    return mask[ei[0], ei[1]][:, None]


def _combine_kernel(nf_ref, a_ref, b_ref, c_ref, o_ref):
    o_ref[...] = (3.0 * nf_ref[...] + a_ref[...] + b_ref[...] + c_ref[...]) / 3.0


def _combine(nf, a, b, c):
    return pl.pallas_call(
        _combine_kernel,
        out_shape=jax.ShapeDtypeStruct((N, D), jnp.float32),
        grid=(8,),
        in_specs=[pl.BlockSpec((N // 8, D), lambda i: (i, 0))] * 4,
        out_specs=pl.BlockSpec((N // 8, D), lambda i: (i, 0)),
    )(nf, a, b, c)


def kernel(edge_index, edgeskip_index, nf, ef, W_s2r, b_s2r, aw_s2r, ab_s2r, W_o2r, b_o2r, aw_o2r, ab_o2r, W_r2s, b_r2s, aw_r2s, ab_r2s, W_r2o, b_r2o, aw_r2o, ab_r2o, W_skip, b_skip, aw_skip, ab_skip):
    ei = edge_index
    es = edgeskip_index
    n = nf.shape[0]
    x_i = nf[ei[1]]
    x_j = nf[ei[0]]
    m1 = jnp.concatenate([x_i, x_j], -1) @ W_s2r + b_s2r
    e1 = _leaky(m1 @ aw_s2r + ab_s2r)
    s2r = ef + _seg_softmax(e1, ei[1], n) * m1
    m2 = jnp.concatenate([x_j, x_i], -1) @ W_o2r + b_o2r
    e2 = _leaky(m2 @ aw_o2r + ab_o2r)
    o2r = ef + _seg_softmax(e2, ei[0], n) * m2
    rel = (s2r + o2r) / 2.0
    m3 = jnp.concatenate([x_j, rel], -1) @ W_r2s + b_r2s
    e3 = _leaky(m3 @ aw_r2s + ab_r2s)
    sub_msg = _seg_softmax(_purify(e3, ei, n, 1, TOPK), ei[0], n) * m3
    m4 = jnp.concatenate([x_i, rel], -1) @ W_r2o + b_r2o
    e4 = _leaky(m4 @ aw_r2o + ab_r2o)
    obj_msg = _seg_softmax(_purify(e4, ei, n, 0, TOPK), ei[1], n) * m4
    xs_i = nf[es[1]]
    xs_j = nf[es[0]]
    m5 = jnp.concatenate([xs_i, xs_j], -1) @ W_skip + b_skip
    e5 = _leaky(m5 @ aw_skip + ab_skip)
    skip_msg = _seg_softmax(_purify(e5, es, n, 0, TOPK), es[1], n) * m5
    sub_agg = jax.ops.segment_sum(sub_msg, ei[0], num_segments=n)
    obj_agg = jax.ops.segment_sum(obj_msg, ei[1], num_segments=n)
    skip_agg = jax.ops.segment_sum(skip_msg, es[1], num_segments=n)
    node = _combine(nf, sub_agg, obj_agg, skip_agg)
    return node, rel


# speedup vs baseline: 1.7190x; 1.7162x over previous
import jax
import jax.numpy as jnp
from jax.experimental import pallas as pl
from jax.experimental.pallas import tpu as pltpu

N = 4096
E = 65536
D = 256
TOPK = 20


def _leaky(x):
    return jax.nn.leaky_relu(x, 0.2)


def _winner_pos(cell):
    perm = jnp.argsort(cell, stable=True).astype(jnp.int32)
    cs = cell[perm]
    starts = jnp.concatenate([jnp.array([True]), cs[1:] != cs[:-1]])
    run_id = jnp.cumsum(starts.astype(jnp.int32)) - 1
    wp_run = jax.ops.segment_max(perm, run_id, num_segments=E)
    return jnp.zeros((E,), jnp.int32).at[perm].set(wp_run[run_id])


def _purify_softmax_w(v, g, winner_pos):
    wv = v[winner_pos]
    is_w = winner_pos == jnp.arange(E, dtype=jnp.int32)
    key2 = jnp.where(is_w, -wv, jnp.inf)
    g_s, _k2, wv_s, isw_s = jax.lax.sort(
        (g, key2, wv, is_w.astype(jnp.int32)), num_keys=2)
    i = jnp.arange(E, dtype=jnp.int32)
    starts = jnp.concatenate([jnp.array([True]), g_s[1:] != g_s[:-1]])
    run_start = jax.lax.associative_scan(jnp.maximum, jnp.where(starts, i, 0))
    pos_in_group = i - run_start
    sel = (pos_in_group == TOPK - 1) & (isw_s == 1)
    thr = jnp.full((N,), -jnp.inf, jnp.float32).at[
        jnp.where(sel, g_s, N)].set(wv_s, mode='drop')
    survive = wv > thr[g]
    P = jnp.where(survive, jnp.exp(wv), 0.0)
    S = jax.ops.segment_sum(P, g, num_segments=N)
    return P / (S[g] + 1e-16)


def _softmax_w(e, idx):
    P = jnp.exp(e)
    S = jax.ops.segment_sum(P, idx, num_segments=N)
    return P / (S[idx] + 1e-16)


def _combine_kernel(nf_ref, a_ref, b_ref, c_ref, o_ref):
    o_ref[...] = (3.0 * nf_ref[...] + a_ref[...] + b_ref[...] + c_ref[...]) / 3.0


def _combine(nf, a, b, c):
    return pl.pallas_call(
        _combine_kernel,
        out_shape=jax.ShapeDtypeStruct((N, D), jnp.float32),
        grid=(8,),
        in_specs=[pl.BlockSpec((N // 8, D), lambda i: (i, 0))] * 4,
        out_specs=pl.BlockSpec((N // 8, D), lambda i: (i, 0)),
    )(nf, a, b, c)


def kernel(edge_index, edgeskip_index, nf, ef, W_s2r, b_s2r, aw_s2r, ab_s2r, W_o2r, b_o2r, aw_o2r, ab_o2r, W_r2s, b_r2s, aw_r2s, ab_r2s, W_r2o, b_r2o, aw_r2o, ab_r2o, W_skip, b_skip, aw_skip, ab_skip):
    ei = edge_index
    es = edgeskip_index
    x_i = nf[ei[1]]
    x_j = nf[ei[0]]
    m1 = x_i @ W_s2r[:D] + x_j @ W_s2r[D:] + b_s2r
    e1 = _leaky(m1 @ aw_s2r + ab_s2r)[:, 0]
    s2r = ef + _softmax_w(e1, ei[1])[:, None] * m1
    m2 = x_j @ W_o2r[:D] + x_i @ W_o2r[D:] + b_o2r
    e2 = _leaky(m2 @ aw_o2r + ab_o2r)[:, 0]
    o2r = ef + _softmax_w(e2, ei[0])[:, None] * m2
    rel = (s2r + o2r) / 2.0
    m3 = x_j @ W_r2s[:D] + rel @ W_r2s[D:] + b_r2s
    e3 = _leaky(m3 @ aw_r2s + ab_r2s)[:, 0]
    m4 = x_i @ W_r2o[:D] + rel @ W_r2o[D:] + b_r2o
    e4 = _leaky(m4 @ aw_r2o + ab_r2o)[:, 0]
    wp_ei = _winner_pos(ei[0] * N + ei[1])
    w3 = _purify_softmax_w(e3, ei[0], wp_ei)
    w4 = _purify_softmax_w(e4, ei[1], wp_ei)
    sub_msg = w3[:, None] * m3
    obj_msg = w4[:, None] * m4
    xs_i = nf[es[1]]
    xs_j = nf[es[0]]
    m5 = xs_i @ W_skip[:D] + xs_j @ W_skip[D:] + b_skip
    e5 = _leaky(m5 @ aw_skip + ab_skip)[:, 0]
    wp_es = _winner_pos(es[0] * N + es[1])
    w5 = _purify_softmax_w(e5, es[1], wp_es)
    skip_msg = w5[:, None] * m5
    sub_agg = jax.ops.segment_sum(sub_msg, ei[0], num_segments=N)
    obj_agg = jax.ops.segment_sum(obj_msg, ei[1], num_segments=N)
    skip_agg = jax.ops.segment_sum(skip_msg, es[1], num_segments=N)
    node = _combine(nf, sub_agg, obj_agg, skip_agg)
    return node, rel


# speedup vs baseline: 1.8504x; 1.0765x over previous
import jax
import jax.numpy as jnp
from jax.experimental import pallas as pl
from jax.experimental.pallas import tpu as pltpu

N = 4096
E = 65536
D = 256
TOPK = 20

_EB = 2048
_GRID = E // _EB


def _leaky(x):
    return jnp.where(x >= 0, x, 0.2 * x)


def _stageA_kernel(xi, xj, xsi, xsj,
                   w1a, w1b, b1, aw1, ab1,
                   w2a, w2b, b2, aw2, ab2,
                   w5a, w5b, b5, aw5, ab5,
                   m1o, m2o, m5o, p1o, p2o, e5o):
    f32 = jnp.float32

    def head(xa, xb, wa, wb, b, aw, ab):
        m = (jnp.dot(xa[...], wa[...], preferred_element_type=f32)
             + jnp.dot(xb[...], wb[...], preferred_element_type=f32)
             + b[...])
        e = _leaky(jnp.sum(m * aw[...], axis=1, keepdims=True) + ab[...])
        return m, e

    m1, e1 = head(xi, xj, w1a, w1b, b1, aw1, ab1)
    m2, e2 = head(xj, xi, w2a, w2b, b2, aw2, ab2)
    m5, e5 = head(xsi, xsj, w5a, w5b, b5, aw5, ab5)
    m1o[...] = m1
    m2o[...] = m2
    m5o[...] = m5
    p1o[...] = jnp.exp(e1)
    p2o[...] = jnp.exp(e2)
    e5o[...] = e5


def _stageA(xi, xj, xsi, xsj, P):
    eb = pl.BlockSpec((_EB, D), lambda i: (i, 0))
    sb = pl.BlockSpec((_EB, 1), lambda i: (i, 0))
    wb = pl.BlockSpec((D, D), lambda i: (0, 0))
    bb = pl.BlockSpec((1, D), lambda i: (0, 0))
    ab = pl.BlockSpec((1, D), lambda i: (0, 0))
    cb = pl.BlockSpec((1, 1), lambda i: (0, 0))
    mshape = jax.ShapeDtypeStruct((E, D), jnp.float32)
    sshape = jax.ShapeDtypeStruct((E, 1), jnp.float32)
    return pl.pallas_call(
        _stageA_kernel,
        grid=(_GRID,),
        in_specs=[eb, eb, eb, eb] + [wb, wb, bb, ab, cb] * 3,
        out_specs=[eb, eb, eb, sb, sb, sb],
        out_shape=[mshape, mshape, mshape, sshape, sshape, sshape],
    )(xi, xj, xsi, xsj,
      P["W_s2r"][:D], P["W_s2r"][D:], P["b_s2r"][None, :], P["aw_s2r"].T, P["ab_s2r"][None, :],
      P["W_o2r"][:D], P["W_o2r"][D:], P["b_o2r"][None, :], P["aw_o2r"].T, P["ab_o2r"][None, :],
      P["W_skip"][:D], P["W_skip"][D:], P["b_skip"][None, :], P["aw_skip"].T, P["ab_skip"][None, :])


def _stageB_kernel(xj, xi, m1, m2, p1, p2, d1, d2, ef,
                   w3a, w3b, b3, aw3, ab3,
                   w4a, w4b, b4, aw4, ab4,
                   relo, m3o, m4o, e3o, e4o):
    f32 = jnp.float32
    w1 = p1[...] / (d1[...] + 1e-16)
    w2 = p2[...] / (d2[...] + 1e-16)
    rel = ef[...] + (w1 * m1[...] + w2 * m2[...]) / 2.0
    relo[...] = rel

    def head(xa, wa, wb, b, aw, ab):
        m = (jnp.dot(xa[...], wa[...], preferred_element_type=f32)
             + jnp.dot(rel, wb[...], preferred_element_type=f32)
             + b[...])
        e = _leaky(jnp.sum(m * aw[...], axis=1, keepdims=True) + ab[...])
        return m, e

    m3, e3 = head(xj, w3a, w3b, b3, aw3, ab3)
    m4, e4 = head(xi, w4a, w4b, b4, aw4, ab4)
    m3o[...] = m3
    m4o[...] = m4
    e3o[...] = e3
    e4o[...] = e4


def _stageB(xj, xi, m1, m2, p1, p2, d1, d2, ef, P):
    eb = pl.BlockSpec((_EB, D), lambda i: (i, 0))
    sb = pl.BlockSpec((_EB, 1), lambda i: (i, 0))
    wb = pl.BlockSpec((D, D), lambda i: (0, 0))
    bb = pl.BlockSpec((1, D), lambda i: (0, 0))
    ab = pl.BlockSpec((1, D), lambda i: (0, 0))
    cb = pl.BlockSpec((1, 1), lambda i: (0, 0))
    mshape = jax.ShapeDtypeStruct((E, D), jnp.float32)
    sshape = jax.ShapeDtypeStruct((E, 1), jnp.float32)
    return pl.pallas_call(
        _stageB_kernel,
        grid=(_GRID,),
        in_specs=[eb, eb, eb, eb, sb, sb, sb, sb, eb] + [wb, wb, bb, ab, cb] * 2,
        out_specs=[eb, eb, eb, sb, sb],
        out_shape=[mshape, mshape, mshape, sshape, sshape],
    )(xj, xi, m1, m2, p1, p2, d1, d2, ef,
      P["W_r2s"][:D], P["W_r2s"][D:], P["b_r2s"][None, :], P["aw_r2s"].T, P["ab_r2s"][None, :],
      P["W_r2o"][:D], P["W_r2o"][D:], P["b_r2o"][None, :], P["aw_r2o"].T, P["ab_r2o"][None, :])


def _combine_kernel(nf_ref, a_ref, b_ref, c_ref, o_ref):
    o_ref[...] = (3.0 * nf_ref[...] + a_ref[...] + b_ref[...] + c_ref[...]) / 3.0


def _combine(nf, a, b, c):
    return pl.pallas_call(
        _combine_kernel,
        out_shape=jax.ShapeDtypeStruct((N, D), jnp.float32),
        grid=(8,),
        in_specs=[pl.BlockSpec((N // 8, D), lambda i: (i, 0))] * 4,
        out_specs=pl.BlockSpec((N // 8, D), lambda i: (i, 0)),
    )(nf, a, b, c)


def _winner_pos(cell):
    perm = jnp.argsort(cell, stable=True).astype(jnp.int32)
    cs = cell[perm]
    starts = jnp.concatenate([jnp.array([True]), cs[1:] != cs[:-1]])
    run_id = jnp.cumsum(starts.astype(jnp.int32)) - 1
    wp_run = jax.ops.segment_max(perm, run_id, num_segments=E)
    return jnp.zeros((E,), jnp.int32).at[perm].set(wp_run[run_id])


def _purify_softmax_w(v, g, winner_pos):
    wv = v[winner_pos]
    is_w = winner_pos == jnp.arange(E, dtype=jnp.int32)
    key2 = jnp.where(is_w, -wv, jnp.inf)
    g_s, _k2, wv_s, isw_s = jax.lax.sort(
        (g, key2, wv, is_w.astype(jnp.int32)), num_keys=2)
    i = jnp.arange(E, dtype=jnp.int32)
    starts = jnp.concatenate([jnp.array([True]), g_s[1:] != g_s[:-1]])
    run_start = jax.lax.associative_scan(jnp.maximum, jnp.where(starts, i, 0))
    pos_in_group = i - run_start
    sel = (pos_in_group == TOPK - 1) & (isw_s == 1)
    thr = jnp.full((N,), -jnp.inf, jnp.float32).at[
        jnp.where(sel, g_s, N)].set(wv_s, mode='drop')
    survive = wv > thr[g]
    P = jnp.where(survive, jnp.exp(wv), 0.0)
    S = jax.ops.segment_sum(P, g, num_segments=N)
    return P / (S[g] + 1e-16)


def kernel(edge_index, edgeskip_index, nf, ef, W_s2r, b_s2r, aw_s2r, ab_s2r, W_o2r, b_o2r, aw_o2r, ab_o2r, W_r2s, b_r2s, aw_r2s, ab_r2s, W_r2o, b_r2o, aw_r2o, ab_r2o, W_skip, b_skip, aw_skip, ab_skip):
    P = {
        "W_s2r": W_s2r, "b_s2r": b_s2r, "aw_s2r": aw_s2r, "ab_s2r": ab_s2r,
        "W_o2r": W_o2r, "b_o2r": b_o2r, "aw_o2r": aw_o2r, "ab_o2r": ab_o2r,
        "W_r2s": W_r2s, "b_r2s": b_r2s, "aw_r2s": aw_r2s, "ab_r2s": ab_r2s,
        "W_r2o": W_r2o, "b_r2o": b_r2o, "aw_r2o": aw_r2o, "ab_r2o": ab_r2o,
        "W_skip": W_skip, "b_skip": b_skip, "aw_skip": aw_skip, "ab_skip": ab_skip,
    }
    ei = edge_index
    es = edgeskip_index
    x_i = nf[ei[1]]
    x_j = nf[ei[0]]
    xs_i = nf[es[1]]
    xs_j = nf[es[0]]

    m1, m2, m5, p1, p2, e5 = _stageA(x_i, x_j, xs_i, xs_j, P)

    p1f = p1[:, 0]
    p2f = p2[:, 0]
    S1 = jax.ops.segment_sum(p1f, ei[1], num_segments=N)
    S2 = jax.ops.segment_sum(p2f, ei[0], num_segments=N)
    d1 = S1[ei[1]][:, None]
    d2 = S2[ei[0]][:, None]

    rel, m3, m4, e3, e4 = _stageB(x_j, x_i, m1, m2, p1, p2, d1, d2, ef, P)

    wp_ei = _winner_pos(ei[0] * N + ei[1])
    w3 = _purify_softmax_w(e3[:, 0], ei[0], wp_ei)
    w4 = _purify_softmax_w(e4[:, 0], ei[1], wp_ei)
    wp_es = _winner_pos(es[0] * N + es[1])
    w5 = _purify_softmax_w(e5[:, 0], es[1], wp_es)

    sub_agg = jax.ops.segment_sum(w3[:, None] * m3, ei[0], num_segments=N)
    obj_agg = jax.ops.segment_sum(w4[:, None] * m4, ei[1], num_segments=N)
    skip_agg = jax.ops.segment_sum(w5[:, None] * m5, es[1], num_segments=N)
    node = _combine(nf, sub_agg, obj_agg, skip_agg)
    return node, rel
